# baseline (speedup 1.0000x reference)
import jax
import jax.numpy as jnp
from jax import lax
from jax.experimental import pallas as pl
from jax.experimental.pallas import tpu as pltpu
from jax.experimental.pallas import tpu_sc as plsc

B = 16384
F = 26
V = 1000000
NC = 2
NS = 16
NW = NC * NS
BPW = B // NW
CHUNK = 128
NCH = BPW // CHUNK
NJ = F * NCH
GRP = 8


def _linearize_body(src, dst, sem):
    pltpu.make_async_copy(src.at[0], dst, sem).start()
    pltpu.make_async_copy(src.at[0], dst, sem).wait()


@jax.jit
def _linearize(w):
    return pl.pallas_call(
        _linearize_body,
        out_shape=jax.ShapeDtypeStruct((V,), jnp.float32),
        in_specs=[pl.BlockSpec(memory_space=pl.ANY)],
        out_specs=pl.BlockSpec(memory_space=pl.ANY),
        scratch_shapes=[pltpu.SemaphoreType.DMA],
    )(w)


def _body(xt_hbm, tab_hbm, bias_hbm, out_hbm, idx_v, buf_v, acc_v, bias_v, sem):
    cid = lax.axis_index("c")
    sid = lax.axis_index("s")
    wid = sid * NC + cid

    pltpu.sync_copy(xt_hbm.at[:, pl.ds(wid * BPW, BPW)], idx_v)
    pltpu.sync_copy(bias_hbm, bias_v)
    binit = bias_v[...]

    def fire(g):
        cps = []
        for jj in range(GRP):
            j = g * GRP + jj
            f, c = j // NCH, j % NCH
            cps.append(
                pltpu.async_copy(
                    tab_hbm.at[idx_v.at[f, pl.ds(c * CHUNK, CHUNK)]],
                    buf_v.at[f, pl.ds(c * CHUNK, CHUNK)],
                    sem,
                )
            )
        return cps

    prev = None
    for g in range(NJ // GRP):
        cur = fire(g)
        if prev is not None:
            for cp in prev:
                cp.wait()
        prev = cur
    for cp in prev:
        cp.wait()

    for g in range(BPW // 16):
        acc16 = binit
        for f in range(F):
            acc16 = acc16 + buf_v[f, pl.ds(g * 16, 16)]
        acc_v[pl.ds(g * 16, 16)] = acc16

    pltpu.sync_copy(acc_v, out_hbm.at[pl.ds(wid * BPW, BPW)])


@jax.jit
def _linear_features(xt, tab, bias):
    mesh = plsc.VectorSubcoreMesh(core_axis_name="c", subcore_axis_name="s")
    return pl.kernel(
        _body,
        out_type=jax.ShapeDtypeStruct((B,), jnp.float32),
        mesh=mesh,
        compiler_params=pltpu.CompilerParams(needs_layout_passes=False),
        scratch_types=[
            pltpu.VMEM((F, BPW), jnp.int32),
            pltpu.VMEM((F, BPW), jnp.float32),
            pltpu.VMEM((BPW,), jnp.float32),
            pltpu.VMEM((16,), jnp.float32),
            pltpu.SemaphoreType.DMA,
        ],
    )(xt, tab, bias)


def kernel(x, fc_weight, bias):
    tab = _linearize(fc_weight.T)
    out = _linear_features(
        x.astype(jnp.int32).T, tab, jnp.broadcast_to(bias, (16,))
    )
    return out.reshape(B, 1)


# speedup vs baseline: 1.0012x; 1.0012x over previous
import jax
import jax.numpy as jnp
from jax import lax
from jax.experimental import pallas as pl
from jax.experimental.pallas import tpu as pltpu
from jax.experimental.pallas import tpu_sc as plsc

B = 16384
F = 26
V = 1000000
NC = 2
NS = 16
NW = NC * NS
BPW = B // NW
CHUNK = 128
NCH = BPW // CHUNK
NJ = F * NCH
GRP = 8


def _linearize_body(src, dst, sem):
    pltpu.make_async_copy(src.at[0], dst, sem).start()
    pltpu.make_async_copy(src.at[0], dst, sem).wait()


@jax.jit
def _linearize(w):
    return pl.pallas_call(
        _linearize_body,
        out_shape=jax.ShapeDtypeStruct((V,), jnp.float32),
        in_specs=[pl.BlockSpec(memory_space=pltpu.MemorySpace.HBM)],
        out_specs=pl.BlockSpec(memory_space=pltpu.MemorySpace.HBM),
        scratch_shapes=[pltpu.SemaphoreType.DMA],
    )(w)


def _body(xt_hbm, tab_hbm, bias_hbm, out_hbm, idx_v, buf_v, acc_v, bias_v, sem):
    cid = lax.axis_index("c")
    sid = lax.axis_index("s")
    wid = sid * NC + cid

    pltpu.sync_copy(xt_hbm.at[:, pl.ds(wid * BPW, BPW)], idx_v)
    pltpu.sync_copy(bias_hbm, bias_v)
    binit = bias_v[...]

    def fire(g):
        cps = []
        for jj in range(GRP):
            j = g * GRP + jj
            f, c = j // NCH, j % NCH
            cps.append(
                pltpu.async_copy(
                    tab_hbm.at[idx_v.at[f, pl.ds(c * CHUNK, CHUNK)]],
                    buf_v.at[f, pl.ds(c * CHUNK, CHUNK)],
                    sem,
                )
            )
        return cps

    prev = None
    for g in range(NJ // GRP):
        cur = fire(g)
        if prev is not None:
            for cp in prev:
                cp.wait()
        prev = cur
    for cp in prev:
        cp.wait()

    for g in range(BPW // 16):
        acc16 = binit
        for f in range(F):
            acc16 = acc16 + buf_v[f, pl.ds(g * 16, 16)]
        acc_v[pl.ds(g * 16, 16)] = acc16

    pltpu.sync_copy(acc_v, out_hbm.at[pl.ds(wid * BPW, BPW)])


@jax.jit
def _linear_features(xt, tab, bias):
    mesh = plsc.VectorSubcoreMesh(core_axis_name="c", subcore_axis_name="s")
    return pl.kernel(
        _body,
        out_type=jax.ShapeDtypeStruct((B,), jnp.float32),
        mesh=mesh,
        compiler_params=pltpu.CompilerParams(needs_layout_passes=False),
        scratch_types=[
            pltpu.VMEM((F, BPW), jnp.int32),
            pltpu.VMEM((F, BPW), jnp.float32),
            pltpu.VMEM((BPW,), jnp.float32),
            pltpu.VMEM((16,), jnp.float32),
            pltpu.SemaphoreType.DMA,
        ],
    )(xt, tab, bias)


def kernel(x, fc_weight, bias):
    tab = _linearize(fc_weight.T)
    out = _linear_features(
        x.astype(jnp.int32).T, tab, jnp.broadcast_to(bias, (16,))
    )
    return out.reshape(B, 1)


# speedup vs baseline: 1.8818x; 1.8795x over previous
import jax
import jax.numpy as jnp
from jax import lax
from jax.experimental import pallas as pl
from jax.experimental.pallas import tpu as pltpu
from jax.experimental.pallas import tpu_sc as plsc

B = 16384
F = 26
V = 1000000
NC = 2
NS = 16
NW = NC * NS
BPW = B // NW
CHUNK = 128
NCH = BPW // CHUNK
NJ = F * NCH
GRP = 8


def _body(xt_hbm, tab_hbm, bias_hbm, out_hbm, idx_v, buf_v, acc_v, bias_v, sem):
    cid = lax.axis_index("c")
    sid = lax.axis_index("s")
    wid = sid * NC + cid

    pltpu.sync_copy(xt_hbm.at[:, pl.ds(wid * BPW, BPW)], idx_v)
    pltpu.sync_copy(bias_hbm, bias_v)
    binit = bias_v[...]

    def fire(g):
        cps = []
        for jj in range(GRP):
            j = g * GRP + jj
            f, c = j // NCH, j % NCH
            cps.append(
                pltpu.async_copy(
                    tab_hbm.at[idx_v.at[f, pl.ds(c * CHUNK, CHUNK)]],
                    buf_v.at[f, pl.ds(c * CHUNK, CHUNK)],
                    sem,
                )
            )
        return cps

    prev = None
    for g in range(NJ // GRP):
        cur = fire(g)
        if prev is not None:
            for cp in prev:
                cp.wait()
        prev = cur
    for cp in prev:
        cp.wait()

    for g in range(BPW // 16):
        acc16 = binit
        for f in range(F):
            acc16 = acc16 + buf_v[f, pl.ds(g * 16, 16)]
        acc_v[pl.ds(g * 16, 16)] = acc16

    pltpu.sync_copy(acc_v, out_hbm.at[pl.ds(wid * BPW, BPW)])


@jax.jit
def _linear_features(xt, tab, bias):
    mesh = plsc.VectorSubcoreMesh(core_axis_name="c", subcore_axis_name="s")
    return pl.kernel(
        _body,
        out_type=jax.ShapeDtypeStruct((B,), jnp.float32),
        mesh=mesh,
        compiler_params=pltpu.CompilerParams(needs_layout_passes=False),
        scratch_types=[
            pltpu.VMEM((F, BPW), jnp.int32),
            pltpu.VMEM((F, BPW), jnp.float32),
            pltpu.VMEM((BPW,), jnp.float32),
            pltpu.VMEM((16,), jnp.float32),
            pltpu.SemaphoreType.DMA,
        ],
    )(xt, tab, bias)


def kernel(x, fc_weight, bias):
    out = _linear_features(
        x.astype(jnp.int32).T,
        fc_weight.reshape(-1),
        jnp.broadcast_to(bias, (16,)),
    )
    return out.reshape(B, 1)


# speedup vs baseline: 2.0770x; 1.1037x over previous
import jax
import jax.numpy as jnp
from jax import lax
from jax.experimental import pallas as pl
from jax.experimental.pallas import tpu as pltpu
from jax.experimental.pallas import tpu_sc as plsc

B = 16384
F = 26
V = 1000000
NC = 2
NS = 16
NW = NC * NS
BPW = B // NW
CHUNK = 128
NCH = BPW // CHUNK
NJ = F * NCH
GRP = 8


LCH = 25000


def _body(
    xt_hbm, tab_hbm, bias_hbm, out_hbm, idx_v, buf_v, acc_v, bias_v, stg_v, spm, sem
):
    cid = lax.axis_index("c")
    sid = lax.axis_index("s")
    wid = sid * NC + cid

    def stage(c):
        pltpu.sync_copy(tab_hbm.at[pl.ds(c * LCH, LCH)], stg_v)
        pltpu.sync_copy(stg_v, spm.at[pl.ds(c * LCH, LCH)])

    stage(sid)
    stage(NS + sid)

    @pl.when(sid < (V // LCH) - 2 * NS)
    def _():
        stage(2 * NS + sid)

    pltpu.sync_copy(xt_hbm.at[:, pl.ds(wid * BPW, BPW)], idx_v)
    pltpu.sync_copy(bias_hbm, bias_v)
    binit = bias_v[...]

    plsc.subcore_barrier()

    def fire(g):
        cps = []
        for jj in range(GRP):
            j = g * GRP + jj
            f, c = j // NCH, j % NCH
            cps.append(
                pltpu.async_copy(
                    spm.at[idx_v.at[f, pl.ds(c * CHUNK, CHUNK)]],
                    buf_v.at[f, pl.ds(c * CHUNK, CHUNK)],
                    sem,
                )
            )
        return cps

    prev = None
    for g in range(NJ // GRP):
        cur = fire(g)
        if prev is not None:
            for cp in prev:
                cp.wait()
        prev = cur
    for cp in prev:
        cp.wait()

    for g in range(BPW // 16):
        acc16 = binit
        for f in range(F):
            acc16 = acc16 + buf_v[f, pl.ds(g * 16, 16)]
        acc_v[pl.ds(g * 16, 16)] = acc16

    pltpu.sync_copy(acc_v, out_hbm.at[pl.ds(wid * BPW, BPW)])


@jax.jit
def _linear_features(xt, tab, bias):
    mesh = plsc.VectorSubcoreMesh(core_axis_name="c", subcore_axis_name="s")
    return pl.kernel(
        _body,
        out_type=jax.ShapeDtypeStruct((B,), jnp.float32),
        mesh=mesh,
        compiler_params=pltpu.CompilerParams(needs_layout_passes=False),
        scratch_types=[
            pltpu.VMEM((F, BPW), jnp.int32),
            pltpu.VMEM((F, BPW), jnp.float32),
            pltpu.VMEM((BPW,), jnp.float32),
            pltpu.VMEM((16,), jnp.float32),
            pltpu.VMEM((LCH,), jnp.float32),
            pltpu.VMEM_SHARED((V,), jnp.float32),
            pltpu.SemaphoreType.DMA,
        ],
    )(xt, tab, bias)


def kernel(x, fc_weight, bias):
    out = _linear_features(
        x.astype(jnp.int32).T,
        fc_weight.reshape(-1),
        jnp.broadcast_to(bias, (16,)),
    )
    return out.reshape(B, 1)


# speedup vs baseline: 4.1994x; 2.0218x over previous
import jax
import jax.numpy as jnp
from jax import lax
from jax.experimental import pallas as pl
from jax.experimental.pallas import tpu as pltpu
from jax.experimental.pallas import tpu_sc as plsc

B = 16384
F = 26
V = 1000000
NC = 2
NS = 16
NW = NC * NS
BPW = B // NW
CHUNK = 128
NCH = BPW // CHUNK
NJ = F * NCH
GRP = 8


LCH = 15616
NST = 4
VT = 64 * LCH


def _body(
    xt_hbm, tab_hbm, tail_hbm, bias_hbm, out_hbm,
    idx_v, buf_v, acc_v, bias_v, stg_v, spm, sem,
):
    cid = lax.axis_index("c")
    sid = lax.axis_index("s")
    wid = sid * NC + cid

    def stage(c):
        off = pl.multiple_of(c * LCH, 1024)
        pltpu.sync_copy(tab_hbm.at[0, pl.ds(off, LCH)], stg_v)
        pltpu.sync_copy(stg_v, spm.at[pl.ds(off, LCH)])

    for k in range(NST):
        stage(k * NS + sid)

    @pl.when(sid == 0)
    def _():
        pltpu.sync_copy(tail_hbm, stg_v.at[pl.ds(0, V - VT)])
        pltpu.sync_copy(stg_v.at[pl.ds(0, V - VT)], spm.at[pl.ds(VT, V - VT)])

    pltpu.sync_copy(xt_hbm.at[:, pl.ds(wid * BPW, BPW)], idx_v)
    pltpu.sync_copy(bias_hbm, bias_v)
    binit = bias_v[...]

    plsc.subcore_barrier()

    def fire(g):
        cps = []
        for jj in range(GRP):
            j = g * GRP + jj
            f, c = j // NCH, j % NCH
            cps.append(
                pltpu.async_copy(
                    spm.at[idx_v.at[f, pl.ds(c * CHUNK, CHUNK)]],
                    buf_v.at[f, pl.ds(c * CHUNK, CHUNK)],
                    sem,
                )
            )
        return cps

    prev = None
    for g in range(NJ // GRP):
        cur = fire(g)
        if prev is not None:
            for cp in prev:
                cp.wait()
        prev = cur
    for cp in prev:
        cp.wait()

    for g in range(BPW // 16):
        acc16 = binit
        for f in range(F):
            acc16 = acc16 + buf_v[f, pl.ds(g * 16, 16)]
        acc_v[pl.ds(g * 16, 16)] = acc16

    pltpu.sync_copy(acc_v, out_hbm.at[pl.ds(wid * BPW, BPW)])


@jax.jit
def _linear_features(xt, tab, tail, bias):
    mesh = plsc.VectorSubcoreMesh(core_axis_name="c", subcore_axis_name="s")
    return pl.kernel(
        _body,
        out_type=jax.ShapeDtypeStruct((B,), jnp.float32),
        mesh=mesh,
        compiler_params=pltpu.CompilerParams(needs_layout_passes=False),
        scratch_types=[
            pltpu.VMEM((F, BPW), jnp.int32),
            pltpu.VMEM((F, BPW), jnp.float32),
            pltpu.VMEM((BPW,), jnp.float32),
            pltpu.VMEM((16,), jnp.float32),
            pltpu.VMEM((LCH,), jnp.float32),
            pltpu.VMEM_SHARED((V,), jnp.float32),
            pltpu.SemaphoreType.DMA,
        ],
    )(xt, tab, tail, bias)


def kernel(x, fc_weight, bias):
    out = _linear_features(
        x.astype(jnp.int32).T,
        fc_weight.T,
        fc_weight[VT:].reshape(-1),
        jnp.broadcast_to(bias, (16,)),
    )
    return out.reshape(B, 1)


# speedup vs baseline: 4.3959x; 1.0468x over previous
import jax
import jax.numpy as jnp
from jax import lax
from jax.experimental import pallas as pl
from jax.experimental.pallas import tpu as pltpu
from jax.experimental.pallas import tpu_sc as plsc

B = 16384
F = 26
V = 1000000
NC = 2
NS = 16
NW = NC * NS
BPW = B // NW
CHUNK = 128
NCH = BPW // CHUNK
NJ = F * NCH
GRP = 8


LCH = 15616
NST = 4
VT = 64 * LCH


def _body(
    xt_hbm, tab_hbm, tail_hbm, bias_hbm, out_hbm,
    idx_v, buf_v, acc_v, bias_v, stg_v, spm, sem,
):
    cid = lax.axis_index("c")
    sid = lax.axis_index("s")
    wid = sid * NC + cid

    def stage(c):
        off = pl.multiple_of(c * LCH, 1024)
        pltpu.sync_copy(tab_hbm.at[0, pl.ds(off, LCH)], spm.at[pl.ds(off, LCH)])

    for k in range(NST):
        stage(k * NS + sid)

    @pl.when(sid == 0)
    def _():
        pltpu.sync_copy(tail_hbm, stg_v.at[pl.ds(0, V - VT)])
        pltpu.sync_copy(stg_v.at[pl.ds(0, V - VT)], spm.at[pl.ds(VT, V - VT)])

    pltpu.sync_copy(xt_hbm.at[:, pl.ds(wid * BPW, BPW)], idx_v)
    pltpu.sync_copy(bias_hbm, bias_v)
    binit = bias_v[...]

    plsc.subcore_barrier()

    def fire(g):
        cps = []
        for jj in range(GRP):
            j = g * GRP + jj
            f, c = j // NCH, j % NCH
            cps.append(
                pltpu.async_copy(
                    spm.at[idx_v.at[f, pl.ds(c * CHUNK, CHUNK)]],
                    buf_v.at[f, pl.ds(c * CHUNK, CHUNK)],
                    sem,
                )
            )
        return cps

    prev = None
    for g in range(NJ // GRP):
        cur = fire(g)
        if prev is not None:
            for cp in prev:
                cp.wait()
        prev = cur
    for cp in prev:
        cp.wait()

    for g in range(BPW // 16):
        acc16 = binit
        for f in range(F):
            acc16 = acc16 + buf_v[f, pl.ds(g * 16, 16)]
        acc_v[pl.ds(g * 16, 16)] = acc16

    pltpu.sync_copy(acc_v, out_hbm.at[pl.ds(wid * BPW, BPW)])


@jax.jit
def _linear_features(xt, tab, tail, bias):
    mesh = plsc.VectorSubcoreMesh(core_axis_name="c", subcore_axis_name="s")
    return pl.kernel(
        _body,
        out_type=jax.ShapeDtypeStruct((B,), jnp.float32),
        mesh=mesh,
        compiler_params=pltpu.CompilerParams(needs_layout_passes=False),
        scratch_types=[
            pltpu.VMEM((F, BPW), jnp.int32),
            pltpu.VMEM((F, BPW), jnp.float32),
            pltpu.VMEM((BPW,), jnp.float32),
            pltpu.VMEM((16,), jnp.float32),
            pltpu.VMEM((LCH,), jnp.float32),
            pltpu.VMEM_SHARED((V,), jnp.float32),
            pltpu.SemaphoreType.DMA,
        ],
    )(xt, tab, tail, bias)


def kernel(x, fc_weight, bias):
    out = _linear_features(
        x.astype(jnp.int32).T,
        fc_weight.T,
        fc_weight[VT:].reshape(-1),
        jnp.broadcast_to(bias, (16,)),
    )
    return out.reshape(B, 1)


# speedup vs baseline: 4.6317x; 1.0536x over previous
import jax
import jax.numpy as jnp
from jax import lax
from jax.experimental import pallas as pl
from jax.experimental.pallas import tpu as pltpu
from jax.experimental.pallas import tpu_sc as plsc

B = 16384
F = 26
V = 1000000
NC = 2
NS = 16
NW = NC * NS
BPW = B // NW
CHUNK = 128
NCH = BPW // CHUNK
NJ = F * NCH
GRP = 8


LCH = 62464
NST = 1
VT = 16 * LCH


def _body(
    xt_hbm, tab_hbm, tail_hbm, bias_hbm, out_hbm,
    idx_v, buf_v, acc_v, bias_v, stg_v, spm, sem,
):
    cid = lax.axis_index("c")
    sid = lax.axis_index("s")
    wid = sid * NC + cid

    cps = []
    for k in range(NST):
        off = pl.multiple_of((k * NS + sid) * LCH, 1024)
        cps.append(
            pltpu.async_copy(
                tab_hbm.at[0, pl.ds(off, LCH)], spm.at[pl.ds(off, LCH)], sem
            )
        )
    cps.append(
        pltpu.async_copy(xt_hbm.at[:, pl.ds(wid * BPW, BPW)], idx_v, sem)
    )
    cps.append(pltpu.async_copy(bias_hbm, bias_v, sem))
    for cp in cps:
        cp.wait()

    @pl.when(sid == 0)
    def _():
        pltpu.sync_copy(tail_hbm, stg_v)
        pltpu.sync_copy(stg_v, spm.at[pl.ds(VT, V - VT)])

    binit = bias_v[...]

    plsc.subcore_barrier()

    def fire(g):
        cps = []
        for jj in range(GRP):
            j = g * GRP + jj
            f, c = j // NCH, j % NCH
            cps.append(
                pltpu.async_copy(
                    spm.at[idx_v.at[f, pl.ds(c * CHUNK, CHUNK)]],
                    buf_v.at[f, pl.ds(c * CHUNK, CHUNK)],
                    sem,
                )
            )
        return cps

    prev = None
    for g in range(NJ // GRP):
        cur = fire(g)
        if prev is not None:
            for cp in prev:
                cp.wait()
        prev = cur
    for cp in prev:
        cp.wait()

    for g in range(BPW // 16):
        acc16 = binit
        for f in range(F):
            acc16 = acc16 + buf_v[f, pl.ds(g * 16, 16)]
        acc_v[pl.ds(g * 16, 16)] = acc16

    pltpu.sync_copy(acc_v, out_hbm.at[pl.ds(wid * BPW, BPW)])


@jax.jit
def _linear_features(xt, tab, tail, bias):
    mesh = plsc.VectorSubcoreMesh(core_axis_name="c", subcore_axis_name="s")
    return pl.kernel(
        _body,
        out_type=jax.ShapeDtypeStruct((B,), jnp.float32),
        mesh=mesh,
        compiler_params=pltpu.CompilerParams(needs_layout_passes=False),
        scratch_types=[
            pltpu.VMEM((F, BPW), jnp.int32),
            pltpu.VMEM((F, BPW), jnp.float32),
            pltpu.VMEM((BPW,), jnp.float32),
            pltpu.VMEM((16,), jnp.float32),
            pltpu.VMEM((V - VT,), jnp.float32),
            pltpu.VMEM_SHARED((V,), jnp.float32),
            pltpu.SemaphoreType.DMA,
        ],
    )(xt, tab, tail, bias)


def kernel(x, fc_weight, bias):
    out = _linear_features(
        x.astype(jnp.int32).T,
        fc_weight.T,
        fc_weight[VT:].reshape(-1),
        jnp.broadcast_to(bias, (16,)),
    )
    return out.reshape(B, 1)


# speedup vs baseline: 4.6672x; 1.0077x over previous
import jax
import jax.numpy as jnp
from jax import lax
from jax.experimental import pallas as pl
from jax.experimental.pallas import tpu as pltpu
from jax.experimental.pallas import tpu_sc as plsc

B = 16384
F = 26
V = 1000000
NC = 2
NS = 16
NW = NC * NS
BPW = B // NW
CHUNK = 128
NCH = BPW // CHUNK
NJ = F * NCH
GRP = 13


LCH = 62464
NST = 1
VT = 16 * LCH


def _body(
    xt_hbm, tab_hbm, tail_hbm, bias_hbm, out_hbm,
    idx_v, buf_v, acc_v, bias_v, stg_v, spm, sem,
):
    cid = lax.axis_index("c")
    sid = lax.axis_index("s")
    wid = sid * NC + cid

    cps = []
    for k in range(NST):
        off = pl.multiple_of((k * NS + sid) * LCH, 1024)
        cps.append(
            pltpu.async_copy(
                tab_hbm.at[0, pl.ds(off, LCH)], spm.at[pl.ds(off, LCH)], sem
            )
        )
    cps.append(
        pltpu.async_copy(xt_hbm.at[:, pl.ds(wid * BPW, BPW)], idx_v, sem)
    )
    cps.append(pltpu.async_copy(bias_hbm, bias_v, sem))
    for cp in cps:
        cp.wait()

    @pl.when(sid == 0)
    def _():
        pltpu.sync_copy(tail_hbm, stg_v)
        pltpu.sync_copy(stg_v, spm.at[pl.ds(VT, V - VT)])

    binit = bias_v[...]

    plsc.subcore_barrier()

    def fire(g):
        cps = []
        for jj in range(GRP):
            j = g * GRP + jj
            f, c = j // NCH, j % NCH
            cps.append(
                pltpu.async_copy(
                    spm.at[idx_v.at[f, pl.ds(c * CHUNK, CHUNK)]],
                    buf_v.at[f, pl.ds(c * CHUNK, CHUNK)],
                    sem,
                )
            )
        return cps

    prev = None
    for g in range(NJ // GRP):
        cur = fire(g)
        if prev is not None:
            for cp in prev:
                cp.wait()
        prev = cur
    for cp in prev:
        cp.wait()

    for g in range(BPW // 16):
        acc16 = binit
        for f in range(F):
            acc16 = acc16 + buf_v[f, pl.ds(g * 16, 16)]
        acc_v[pl.ds(g * 16, 16)] = acc16

    pltpu.sync_copy(acc_v, out_hbm.at[pl.ds(wid * BPW, BPW)])


@jax.jit
def _linear_features(xt, tab, tail, bias):
    mesh = plsc.VectorSubcoreMesh(core_axis_name="c", subcore_axis_name="s")
    return pl.kernel(
        _body,
        out_type=jax.ShapeDtypeStruct((B,), jnp.float32),
        mesh=mesh,
        compiler_params=pltpu.CompilerParams(needs_layout_passes=False),
        scratch_types=[
            pltpu.VMEM((F, BPW), jnp.int32),
            pltpu.VMEM((F, BPW), jnp.float32),
            pltpu.VMEM((BPW,), jnp.float32),
            pltpu.VMEM((16,), jnp.float32),
            pltpu.VMEM((V - VT,), jnp.float32),
            pltpu.VMEM_SHARED((V,), jnp.float32),
            pltpu.SemaphoreType.DMA,
        ],
    )(xt, tab, tail, bias)


def kernel(x, fc_weight, bias):
    out = _linear_features(
        x.astype(jnp.int32).T,
        fc_weight.T,
        fc_weight[VT:].reshape(-1),
        jnp.broadcast_to(bias, (16,)),
    )
    return out.reshape(B, 1)
